# baseline (speedup 1.0000x reference)
import functools
import numpy as np
import jax
import jax.numpy as jnp
from jax.experimental import pallas as pl
from jax.experimental.pallas import tpu as pltpu

N = 10000
E = 320000
H = 128
RSQRT_H = 1.0 / np.sqrt(128.0)


def _edge_pass_conv(q, k, v, qW, s, ea, src, dst):
    alpha = (jnp.sum(q[dst] * k[src], axis=1) + jnp.sum(qW[dst] * ea, axis=1)) * RSQRT_H
    ex = jnp.exp(alpha - s[dst])
    denom = jax.ops.segment_sum(ex, dst, num_segments=N)
    acc1 = jax.ops.segment_sum(ex[:, None] * v[src], dst, num_segments=N)
    acc2 = jax.ops.segment_sum(ex[:, None] * ea, dst, num_segments=N)
    return denom, acc1, acc2


def _conv(h, src, dst, ea, p, ea_norm_max):
    q = h @ p["Wq"] + p["bq"]
    k = h @ p["Wk"] + p["bk"]
    v = h @ p["Wv"] + p["bv"]
    qW = q @ p["We"].T
    K = jnp.max(jnp.linalg.norm(k, axis=1))
    s = (jnp.linalg.norm(q, axis=1) * K
         + jnp.linalg.norm(qW, axis=1) * ea_norm_max) * RSQRT_H
    denom, acc1, acc2 = _edge_pass_conv(q, k, v, qW, s, ea, src, dst)
    num = acc1 + acc2 @ p["We"]
    out = jnp.where(denom[:, None] > 0, num / denom[:, None], 0.0)
    return out + h @ p["Ws"] + p["bs"]


def _sigmoid_kernel(x_ref, o_ref):
    o_ref[...] = jax.nn.sigmoid(x_ref[...])


def _pallas_sigmoid(x):
    blk = 20000
    return pl.pallas_call(
        _sigmoid_kernel,
        grid=(x.shape[0] // blk,),
        in_specs=[pl.BlockSpec((blk, 1), lambda i: (i, 0))],
        out_specs=pl.BlockSpec((blk, 1), lambda i: (i, 0)),
        out_shape=jax.ShapeDtypeStruct(x.shape, x.dtype),
    )(x)


def kernel(x, edge_attr, edge_index, params):
    p = params
    src, dst = edge_index[0], edge_index[1]
    h = x @ p["node_W"] + p["node_b"]
    ea1 = edge_attr @ p["edge_W"] + p["edge_b"]
    A1 = jnp.max(jnp.linalg.norm(ea1, axis=1))
    h = jax.nn.leaky_relu(_conv(h, src, dst, ea1, p["conv1"], A1))
    hs = h @ p["eu1_W"][:H]
    hd = h @ p["eu1_W"][H:2 * H]
    ea2 = jax.nn.leaky_relu(hs[src] + hd[dst] + ea1 @ p["eu1_W"][2 * H:] + p["eu1_b"])
    A2 = jnp.max(jnp.linalg.norm(ea2, axis=1))
    h = jax.nn.leaky_relu(_conv(h, src, dst, ea2, p["conv2"], A2))
    hs = h @ p["eu2_W"][:H]
    hd = h @ p["eu2_W"][H:2 * H]
    ea3 = jax.nn.leaky_relu(hs[src] + hd[dst] + ea2 @ p["eu2_W"][2 * H:] + p["eu2_b"])
    A3 = jnp.max(jnp.linalg.norm(ea3, axis=1))
    h = _conv(h, src, dst, ea3, p["conv3"], A3)
    w = p["eu3_W"] @ p["fc_W"]
    c = p["eu3_b"] @ p["fc_W"] + p["fc_b"]
    g1 = h @ w[:H]
    g2 = h @ w[H:2 * H]
    z = g1[src] + g2[dst] + ea3 @ w[2 * H:] + c
    return _pallas_sigmoid(z)


# speedup vs baseline: 2.3663x; 2.3663x over previous
import functools
import numpy as np
import jax
import jax.numpy as jnp
from jax import lax
from jax.experimental import pallas as pl
from jax.experimental.pallas import tpu as pltpu
from jax.experimental.pallas import tpu_sc as plsc

N = 10000
E = 320000
H = 128
RSQRT_H = float(1.0 / np.sqrt(128.0))

NC = 2
NS = 16
NW = NC * NS
EPW = E // NW
C = 80
NCHUNK = EPW // C
NP = 10112
HQ = 144
NPT = NP // NS

@functools.cache
def _mesh():
    return plsc.VectorSubcoreMesh(core_axis_name="c", subcore_axis_name="s",
                                  num_cores=NC, num_subcores=NS)


def _conv_edge_body(src_h, dst_h, q_h, kv_h, ew_h, z128_h, z1_h,
                    acc_o, den_o,
                    sidx, didx, qb, kvb, ewb, exb,
                    acc_sp, den_sp, sem):
    cid = lax.axis_index("c")
    sid = lax.axis_index("s")
    wid = cid * NS + sid
    row0 = sid * NPT
    pltpu.sync_copy(z128_h.at[pl.ds(row0, NPT)], acc_sp.at[pl.ds(row0, NPT)])
    pltpu.sync_copy(z1_h.at[pl.ds(row0, NPT)], den_sp.at[pl.ds(row0, NPT)])
    plsc.subcore_barrier()

    def chunk(ci, carry):
        base = wid * EPW + ci * C
        pltpu.sync_copy(src_h.at[pl.ds(base, C)], sidx)
        pltpu.sync_copy(dst_h.at[pl.ds(base, C)], didx)
        cp_q = pltpu.async_copy(q_h.at[didx], qb, sem)
        cp_kv = pltpu.async_copy(kv_h.at[sidx], kvb, sem)
        cp_ew = pltpu.async_copy(ew_h.at[pl.ds(base, C)], ewb, sem)
        cp_q.wait()
        cp_kv.wait()
        cp_ew.wait()

        lanes = lax.iota(jnp.int32, 16)

        perms = [lanes ^ sh for sh in (8, 4, 2, 1)]

        def group(g, c2):
            av = jnp.zeros((16,), jnp.float32)
            for j in range(16):
                e = g * 16 + j
                acc = jnp.zeros((16,), jnp.float32)
                for t in range(8):
                    sl = pl.ds(t * 16, 16)
                    acc = acc + qb[e, sl] * (kvb[e, sl] + ewb[e, sl])
                acc = acc + qb[e, pl.ds(128, 16)]
                for perm in perms:
                    acc = acc + acc.at[perm].get(mode="promise_in_bounds")
                av = jnp.where(lanes == j, acc, av)
            sl16 = pl.ds(g * 16, 16)
            exv = jnp.exp(av)
            exb[sl16] = exv
            for j in range(16):
                e = g * 16 + j
                x = exv[j]
                for t in range(8):
                    ewb[e, pl.ds(t * 16, 16)] = x * (
                        kvb[e, pl.ds(128 + t * 16, 16)] + ewb[e, pl.ds(t * 16, 16)])
            return c2

        lax.fori_loop(0, C // 16, group, 0, unroll=False)

        pltpu.sync_copy(ewb, acc_sp.at[didx], add=True)
        pltpu.sync_copy(exb, den_sp.at[didx], add=True)
        return carry

    lax.fori_loop(0, NCHUNK, chunk, 0, unroll=False)
    plsc.subcore_barrier()
    pltpu.sync_copy(acc_sp.at[pl.ds(row0, NPT)], acc_o.at[cid, pl.ds(row0, NPT)])
    pltpu.sync_copy(den_sp.at[pl.ds(row0, NPT)], den_o.at[cid, pl.ds(row0, NPT)])


@functools.cache
def _conv_edge_pass():
    return pl.kernel(
        _conv_edge_body,
        out_type=[jax.ShapeDtypeStruct((NC, NP, H), jnp.float32),
                  jax.ShapeDtypeStruct((NC, NP), jnp.float32)],
        mesh=_mesh(),
        compiler_params=pltpu.CompilerParams(use_tc_tiling_on_sc=False,
                                             needs_layout_passes=False),
        scratch_types=[
        pltpu.VMEM((C,), jnp.int32),
        pltpu.VMEM((C,), jnp.int32),
        pltpu.VMEM((C, HQ), jnp.float32),
        pltpu.VMEM((C, 2 * H), jnp.float32),
        pltpu.VMEM((C, H), jnp.float32),
        pltpu.VMEM((C,), jnp.float32),
        pltpu.VMEM_SHARED((NP, H), jnp.float32),
        pltpu.VMEM_SHARED((NP,), jnp.float32),
        pltpu.SemaphoreType.DMA,
        ],
    )


def _conv(h, src, dst, eaW, AW, p):
    q = h @ p["Wq"] + p["bq"]
    k = h @ p["Wk"] + p["bk"]
    v = h @ p["Wv"] + p["bv"]
    kv = jnp.concatenate([k, v], axis=1)
    K = jnp.max(jnp.linalg.norm(k, axis=1))
    s = jnp.linalg.norm(q, axis=1) * (K + AW) * RSQRT_H
    qpad = jnp.concatenate(
        [q * RSQRT_H, -s[:, None], jnp.zeros((N, HQ - H - 1), jnp.float32)], axis=1)
    z128 = jnp.zeros((NP, H), jnp.float32)
    z1 = jnp.zeros((NP,), jnp.float32)
    acc, den = _conv_edge_pass()(src, dst, qpad, kv, eaW, z128, z1)
    num = acc[0, :N] + acc[1, :N]
    denom = den[0, :N] + den[1, :N]
    out = jnp.where(denom[:, None] > 0, num / denom[:, None], 0.0)
    return out + h @ p["Ws"] + p["bs"]


def kernel(x, edge_attr, edge_index, params):
    p = params
    src, dst = edge_index[0], edge_index[1]
    h = x @ p["node_W"] + p["node_b"]

    c1 = p["conv1"]
    eaW1 = edge_attr @ (p["edge_W"] @ c1["We"]) + p["edge_b"] @ c1["We"]
    AW1 = jnp.max(jnp.linalg.norm(eaW1, axis=1))
    h = jax.nn.leaky_relu(_conv(h, src, dst, eaW1, AW1, c1))

    ea1 = edge_attr @ p["edge_W"] + p["edge_b"]
    hs = h @ p["eu1_W"][:H]
    hd = h @ p["eu1_W"][H:2 * H]
    ea2 = jax.nn.leaky_relu(hs[src] + hd[dst] + ea1 @ p["eu1_W"][2 * H:] + p["eu1_b"])

    c2 = p["conv2"]
    eaW2 = ea2 @ c2["We"]
    AW2 = jnp.max(jnp.linalg.norm(eaW2, axis=1))
    h = jax.nn.leaky_relu(_conv(h, src, dst, eaW2, AW2, c2))

    hs = h @ p["eu2_W"][:H]
    hd = h @ p["eu2_W"][H:2 * H]
    ea3 = jax.nn.leaky_relu(hs[src] + hd[dst] + ea2 @ p["eu2_W"][2 * H:] + p["eu2_b"])

    c3 = p["conv3"]
    eaW3 = ea3 @ c3["We"]
    AW3 = jnp.max(jnp.linalg.norm(eaW3, axis=1))
    h = _conv(h, src, dst, eaW3, AW3, c3)

    w = p["eu3_W"] @ p["fc_W"]
    c = p["eu3_b"] @ p["fc_W"] + p["fc_b"]
    g1 = h @ w[:H]
    g2 = h @ w[H:2 * H]
    z = g1[src] + g2[dst] + ea3 @ w[2 * H:] + c
    return jax.nn.sigmoid(z)


# speedup vs baseline: 4.6887x; 1.9815x over previous
import functools
import numpy as np
import jax
import jax.numpy as jnp
from jax import lax
from jax.experimental import pallas as pl
from jax.experimental.pallas import tpu as pltpu
from jax.experimental.pallas import tpu_sc as plsc

N = 10000
E = 320000
H = 128
RSQRT_H = float(1.0 / np.sqrt(128.0))

NC = 2
NS = 16
NW = NC * NS
EPW = E // NW
C = 80
NCHUNK = EPW // C
NP = 10112
HQ = 144
NPT = NP // NS

@functools.cache
def _mesh():
    return plsc.VectorSubcoreMesh(core_axis_name="c", subcore_axis_name="s",
                                  num_cores=NC, num_subcores=NS)


def _conv_edge_body(src_h, dst_h, q_h, kv_h, ew_h, z128_h, z1_h,
                    acc_o, den_o,
                    sidx, didx, qb, kvb, ewb, exb,
                    acc_sp, den_sp, sem):
    cid = lax.axis_index("c")
    sid = lax.axis_index("s")
    wid = cid * NS + sid
    row0 = sid * NPT
    pltpu.sync_copy(z128_h.at[pl.ds(row0, NPT)], acc_sp.at[pl.ds(row0, NPT)])
    pltpu.sync_copy(z1_h.at[pl.ds(row0, NPT)], den_sp.at[pl.ds(row0, NPT)])
    plsc.subcore_barrier()

    def chunk(ci, carry):
        base = wid * EPW + ci * C
        pltpu.sync_copy(src_h.at[pl.ds(base, C)], sidx)
        pltpu.sync_copy(dst_h.at[pl.ds(base, C)], didx)
        cp_q = pltpu.async_copy(q_h.at[didx], qb, sem)
        cp_kv = pltpu.async_copy(kv_h.at[sidx], kvb, sem)
        cp_ew = pltpu.async_copy(ew_h.at[pl.ds(base, C)], ewb, sem)
        cp_q.wait()
        cp_kv.wait()
        cp_ew.wait()

        lanes = lax.iota(jnp.int32, 16)

        perms = [lanes ^ sh for sh in (8, 4, 2, 1)]

        def group(g, c2):
            av = jnp.zeros((16,), jnp.float32)
            for j in range(16):
                e = g * 16 + j
                acc = jnp.zeros((16,), jnp.float32)
                for t in range(8):
                    sl = pl.ds(t * 16, 16)
                    acc = acc + qb[e, sl] * (kvb[e, sl] + ewb[e, sl])
                acc = acc + qb[e, pl.ds(128, 16)]
                for perm in perms:
                    acc = acc + acc.at[perm].get(mode="promise_in_bounds")
                av = jnp.where(lanes == j, acc, av)
            sl16 = pl.ds(g * 16, 16)
            exv = jnp.exp(av)
            exb[sl16] = exv
            for j in range(16):
                e = g * 16 + j
                x = exv[j]
                for t in range(8):
                    ewb[e, pl.ds(t * 16, 16)] = x * (
                        kvb[e, pl.ds(128 + t * 16, 16)] + ewb[e, pl.ds(t * 16, 16)])
            return c2

        lax.fori_loop(0, C // 16, group, 0, unroll=False)

        pltpu.sync_copy(ewb, acc_sp.at[didx], add=True)
        pltpu.sync_copy(exb, den_sp.at[didx], add=True)
        return carry

    lax.fori_loop(0, NCHUNK, chunk, 0, unroll=False)
    plsc.subcore_barrier()
    pltpu.sync_copy(acc_sp.at[pl.ds(row0, NPT)], acc_o.at[cid, pl.ds(row0, NPT)])
    pltpu.sync_copy(den_sp.at[pl.ds(row0, NPT)], den_o.at[cid, pl.ds(row0, NPT)])


@functools.cache
def _conv_edge_pass():
    return pl.kernel(
        _conv_edge_body,
        out_type=[jax.ShapeDtypeStruct((NC, NP, H), jnp.float32),
                  jax.ShapeDtypeStruct((NC, NP), jnp.float32)],
        mesh=_mesh(),
        compiler_params=pltpu.CompilerParams(use_tc_tiling_on_sc=False,
                                             needs_layout_passes=False),
        scratch_types=[
        pltpu.VMEM((C,), jnp.int32),
        pltpu.VMEM((C,), jnp.int32),
        pltpu.VMEM((C, HQ), jnp.float32),
        pltpu.VMEM((C, 2 * H), jnp.float32),
        pltpu.VMEM((C, H), jnp.float32),
        pltpu.VMEM((C,), jnp.float32),
        pltpu.VMEM_SHARED((NP, H), jnp.float32),
        pltpu.VMEM_SHARED((NP,), jnp.float32),
        pltpu.SemaphoreType.DMA,
        ],
    )


def _eu_body_with_dvec(src_h, dst_h, a_h, b_h, t_h, wv_h,
                       ea_o, rm_o, dv_o,
                       sidx, didx, abf, bbf, tbf, dvb, rmb, wvb, sem):
    cid = lax.axis_index("c")
    sid = lax.axis_index("s")
    wid = cid * NS + sid
    pltpu.sync_copy(wv_h, wvb)
    lanes = lax.iota(jnp.int32, 16)
    perms = [lanes ^ sh for sh in (8, 4, 2, 1)]

    def chunk(ci, rm):
        base = wid * EPW + ci * C
        pltpu.sync_copy(src_h.at[pl.ds(base, C)], sidx)
        pltpu.sync_copy(dst_h.at[pl.ds(base, C)], didx)
        cp_a = pltpu.async_copy(a_h.at[sidx], abf, sem)
        cp_b = pltpu.async_copy(b_h.at[didx], bbf, sem)
        cp_t = pltpu.async_copy(t_h.at[pl.ds(base, C)], tbf, sem)
        cp_a.wait()
        cp_b.wait()
        cp_t.wait()

        def group(g, rm2):
            dvv = jnp.zeros((16,), jnp.float32)
            for j in range(16):
                e = g * 16 + j
                sq = jnp.zeros((16,), jnp.float32)
                dv = jnp.zeros((16,), jnp.float32)
                for t in range(8):
                    sl = pl.ds(t * 16, 16)
                    z = abf[e, sl] + bbf[e, sl] + tbf[e, sl]
                    z = jnp.where(z > 0, z, z * jnp.float32(0.01))
                    tbf[e, sl] = z
                    sq = sq + z * z
                    dv = dv + z * wvb[sl]
                for perm in perms:
                    sq = sq + sq.at[perm].get(mode="promise_in_bounds")
                    dv = dv + dv.at[perm].get(mode="promise_in_bounds")
                rm2 = jnp.maximum(rm2, sq)
                dvv = jnp.where(lanes == j, dv, dvv)
            dvb[pl.ds(g * 16, 16)] = dvv
            return rm2

        rm = lax.fori_loop(0, C // 16, group, rm, unroll=False)
        pltpu.sync_copy(tbf, ea_o.at[pl.ds(base, C)])
        pltpu.sync_copy(dvb, dv_o.at[pl.ds(base, C)])
        return rm

    rm = lax.fori_loop(0, NCHUNK, chunk, jnp.zeros((16,), jnp.float32),
                       unroll=False)
    rmb[...] = rm
    pltpu.sync_copy(rmb, rm_o.at[cid, sid])


@functools.cache
def _eu_pass():
    return pl.kernel(
        _eu_body_with_dvec,
        out_type=[jax.ShapeDtypeStruct((E, H), jnp.float32),
                  jax.ShapeDtypeStruct((NC, NS, 16), jnp.float32),
                  jax.ShapeDtypeStruct((E,), jnp.float32)],
        mesh=_mesh(),
        compiler_params=pltpu.CompilerParams(use_tc_tiling_on_sc=False,
                                             needs_layout_passes=False),
        scratch_types=[
            pltpu.VMEM((C,), jnp.int32),
            pltpu.VMEM((C,), jnp.int32),
            pltpu.VMEM((C, H), jnp.float32),
            pltpu.VMEM((C, H), jnp.float32),
            pltpu.VMEM((C, H), jnp.float32),
            pltpu.VMEM((C,), jnp.float32),
            pltpu.VMEM((16,), jnp.float32),
            pltpu.VMEM((H,), jnp.float32),
            pltpu.SemaphoreType.DMA,
        ],
    )


def _final_body(src_h, dst_h, g1_h, g2_h, dv_h, out_o,
                sidx, didx, g1b, g2b, dvb, ob, sem):
    cid = lax.axis_index("c")
    sid = lax.axis_index("s")
    wid = cid * NS + sid

    def chunk(ci, carry):
        base = wid * EPW + ci * C
        pltpu.sync_copy(src_h.at[pl.ds(base, C)], sidx)
        pltpu.sync_copy(dst_h.at[pl.ds(base, C)], didx)
        cp_a = pltpu.async_copy(g1_h.at[sidx], g1b, sem)
        cp_b = pltpu.async_copy(g2_h.at[didx], g2b, sem)
        cp_d = pltpu.async_copy(dv_h.at[pl.ds(base, C)], dvb, sem)
        cp_a.wait()
        cp_b.wait()
        cp_d.wait()

        def group(g, c2):
            sl = pl.ds(g * 16, 16)
            z = g1b[sl] + g2b[sl] + dvb[sl]
            ob[sl] = 1.0 / (1.0 + jnp.exp(-z))
            return c2

        lax.fori_loop(0, C // 16, group, 0, unroll=False)
        pltpu.sync_copy(ob, out_o.at[pl.ds(base, C)])
        return carry

    lax.fori_loop(0, NCHUNK, chunk, 0, unroll=False)


@functools.cache
def _final_pass():
    return pl.kernel(
        _final_body,
        out_type=jax.ShapeDtypeStruct((E,), jnp.float32),
        mesh=_mesh(),
        compiler_params=pltpu.CompilerParams(use_tc_tiling_on_sc=False,
                                             needs_layout_passes=False),
        scratch_types=[
            pltpu.VMEM((C,), jnp.int32),
            pltpu.VMEM((C,), jnp.int32),
            pltpu.VMEM((C,), jnp.float32),
            pltpu.VMEM((C,), jnp.float32),
            pltpu.VMEM((C,), jnp.float32),
            pltpu.VMEM((C,), jnp.float32),
            pltpu.SemaphoreType.DMA,
        ],
    )


def _conv(h, src, dst, eaW, AW, p):
    q = h @ p["Wq"] + p["bq"]
    k = h @ p["Wk"] + p["bk"]
    v = h @ p["Wv"] + p["bv"]
    kv = jnp.concatenate([k, v], axis=1)
    K = jnp.max(jnp.linalg.norm(k, axis=1))
    s = jnp.linalg.norm(q, axis=1) * (K + AW) * RSQRT_H
    qpad = jnp.concatenate(
        [q * RSQRT_H, -s[:, None], jnp.zeros((N, HQ - H - 1), jnp.float32)], axis=1)
    z128 = jnp.zeros((NP, H), jnp.float32)
    z1 = jnp.zeros((NP,), jnp.float32)
    acc, den = _conv_edge_pass()(src, dst, qpad, kv, eaW, z128, z1)
    num = acc[0, :N] + acc[1, :N]
    denom = den[0, :N] + den[1, :N]
    out = jnp.where(denom[:, None] > 0, num / denom[:, None], 0.0)
    return out + h @ p["Ws"] + p["bs"]


def kernel(x, edge_attr, edge_index, params):
    p = params
    src, dst = edge_index[0], edge_index[1]
    h = x @ p["node_W"] + p["node_b"]

    c1 = p["conv1"]
    eaW1 = edge_attr @ (p["edge_W"] @ c1["We"]) + p["edge_b"] @ c1["We"]
    AW1 = jnp.max(jnp.linalg.norm(eaW1, axis=1))
    h = jax.nn.leaky_relu(_conv(h, src, dst, eaW1, AW1, c1))

    w = p["eu3_W"] @ p["fc_W"]
    c = p["eu3_b"] @ p["fc_W"] + p["fc_b"]

    A1 = h @ p["eu1_W"][:H]
    B1 = h @ p["eu1_W"][H:2 * H]
    T1 = edge_attr @ (p["edge_W"] @ p["eu1_W"][2 * H:]) + (
        p["edge_b"] @ p["eu1_W"][2 * H:] + p["eu1_b"])
    ea2, rm1, _ = _eu_pass()(src, dst, A1, B1, T1, jnp.zeros((H,), jnp.float32))
    Aea2 = jnp.sqrt(jnp.max(rm1))

    c2 = p["conv2"]
    eaW2 = ea2 @ c2["We"]
    AW2 = Aea2 * jnp.linalg.norm(c2["We"])
    h = jax.nn.leaky_relu(_conv(h, src, dst, eaW2, AW2, c2))

    A2 = h @ p["eu2_W"][:H]
    B2 = h @ p["eu2_W"][H:2 * H]
    T2 = ea2 @ p["eu2_W"][2 * H:] + p["eu2_b"]
    ea3, rm2, dvec = _eu_pass()(src, dst, A2, B2, T2, w[2 * H:, 0])
    Aea3 = jnp.sqrt(jnp.max(rm2))

    c3 = p["conv3"]
    eaW3 = ea3 @ c3["We"]
    AW3 = Aea3 * jnp.linalg.norm(c3["We"])
    h = _conv(h, src, dst, eaW3, AW3, c3)

    g1 = (h @ w[:H])[:, 0]
    g2 = (h @ w[H:2 * H])[:, 0] + c[0]
    out = _final_pass()(src, dst, g1, g2, dvec)
    return out[:, None]


# speedup vs baseline: 5.2043x; 1.1100x over previous
import functools
import numpy as np
import jax
import jax.numpy as jnp
from jax import lax
from jax.experimental import pallas as pl
from jax.experimental.pallas import tpu as pltpu
from jax.experimental.pallas import tpu_sc as plsc

N = 10000
E = 320000
H = 128
RSQRT_H = float(1.0 / np.sqrt(128.0))

NC = 2
NS = 16
NW = NC * NS
EPW = E // NW
C = 80
NCHUNK = EPW // C
NP = 10112
HQ = 144
NPT = NP // NS

@functools.cache
def _mesh():
    return plsc.VectorSubcoreMesh(core_axis_name="c", subcore_axis_name="s",
                                  num_cores=NC, num_subcores=NS)


def _conv_edge_body(src_h, dst_h, q_h, kv_h, ew_h, z128_h, z1_h,
                    acc_o, den_o,
                    sidx2, didx2, qb, kvb, ewb, exb,
                    acc_sp, den_sp,
                    sem_q, sem_kv, sem_ew, sem_sc, sem_ix):
    cid = lax.axis_index("c")
    sid = lax.axis_index("s")
    wid = cid * NS + sid
    row0 = sid * NPT
    pltpu.sync_copy(z128_h.at[pl.ds(row0, NPT)], acc_sp.at[pl.ds(row0, NPT)])
    pltpu.sync_copy(z1_h.at[pl.ds(row0, NPT)], den_sp.at[pl.ds(row0, NPT)])
    plsc.subcore_barrier()

    e0 = wid * EPW
    pltpu.sync_copy(src_h.at[pl.ds(e0, C)], sidx2.at[0])
    pltpu.sync_copy(dst_h.at[pl.ds(e0, C)], didx2.at[0])
    pltpu.async_copy(q_h.at[didx2.at[0]], qb, sem_q)
    pltpu.async_copy(kv_h.at[sidx2.at[0]], kvb, sem_kv)
    pltpu.async_copy(ew_h.at[pl.ds(e0, C)], ewb, sem_ew)
    pltpu.async_copy(src_h.at[pl.ds(e0 + C, C)], sidx2.at[1], sem_ix)
    pltpu.async_copy(dst_h.at[pl.ds(e0 + C, C)], didx2.at[1], sem_ix)

    lanes = lax.iota(jnp.int32, 16)
    perms = [lanes ^ sh for sh in (8, 4, 2, 1)]

    def compute(cur):
        def group(g, c2):
            av = jnp.zeros((16,), jnp.float32)
            for j in range(16):
                e = g * 16 + j
                acc = jnp.zeros((16,), jnp.float32)
                for t in range(8):
                    sl = pl.ds(t * 16, 16)
                    acc = acc + qb[e, sl] * (kvb[e, sl] + ewb[e, sl])
                acc = acc + qb[e, pl.ds(128, 16)]
                for perm in perms:
                    acc = acc + acc.at[perm].get(mode="promise_in_bounds")
                av = jnp.where(lanes == j, acc, av)
            sl16 = pl.ds(g * 16, 16)
            exv = jnp.exp(av)
            exb[sl16] = exv
            for j in range(16):
                e = g * 16 + j
                x = exv[j]
                for t in range(8):
                    ewb[e, pl.ds(t * 16, 16)] = x * (
                        kvb[e, pl.ds(128 + t * 16, 16)] + ewb[e, pl.ds(t * 16, 16)])
            return c2
        lax.fori_loop(0, C // 16, group, 0, unroll=False)

    def iter_body(ci, carry):
        cur = lax.rem(ci, 2)
        nxt = 1 - cur
        base = e0 + ci * C
        pltpu.make_async_copy(q_h.at[didx2.at[cur]], qb, sem_q).wait()
        pltpu.make_async_copy(kv_h.at[sidx2.at[cur]], kvb, sem_kv).wait()
        pltpu.make_async_copy(ew_h.at[pl.ds(base, C)], ewb, sem_ew).wait()
        compute(cur)
        cp_acc = pltpu.async_copy(ewb, acc_sp.at[didx2.at[cur]], sem_sc, add=True)
        cp_den = pltpu.async_copy(exb, den_sp.at[didx2.at[cur]], sem_sc, add=True)
        nbase = e0 + (ci + 1) * C

        @pl.when(ci + 1 < NCHUNK)
        def _():
            pltpu.make_async_copy(src_h.at[pl.ds(nbase, C)], sidx2.at[nxt], sem_ix).wait()
            pltpu.make_async_copy(dst_h.at[pl.ds(nbase, C)], didx2.at[nxt], sem_ix).wait()
            pltpu.async_copy(q_h.at[didx2.at[nxt]], qb, sem_q)
            pltpu.async_copy(kv_h.at[sidx2.at[nxt]], kvb, sem_kv)

        cp_acc.wait()
        cp_den.wait()

        @pl.when(ci + 1 < NCHUNK)
        def _():
            pltpu.async_copy(ew_h.at[pl.ds(nbase, C)], ewb, sem_ew)

        @pl.when(ci + 2 < NCHUNK)
        def _():
            pbase = e0 + (ci + 2) * C
            pltpu.async_copy(src_h.at[pl.ds(pbase, C)], sidx2.at[cur], sem_ix)
            pltpu.async_copy(dst_h.at[pl.ds(pbase, C)], didx2.at[cur], sem_ix)

        return carry

    lax.fori_loop(0, NCHUNK, iter_body, 0, unroll=False)

    plsc.subcore_barrier()
    pltpu.sync_copy(acc_sp.at[pl.ds(row0, NPT)], acc_o.at[cid, pl.ds(row0, NPT)])
    pltpu.sync_copy(den_sp.at[pl.ds(row0, NPT)], den_o.at[cid, pl.ds(row0, NPT)])


@functools.cache
def _conv_edge_pass():
    return pl.kernel(
        _conv_edge_body,
        out_type=[jax.ShapeDtypeStruct((NC, NP, H), jnp.float32),
                  jax.ShapeDtypeStruct((NC, NP), jnp.float32)],
        mesh=_mesh(),
        compiler_params=pltpu.CompilerParams(use_tc_tiling_on_sc=False,
                                             needs_layout_passes=False),
        scratch_types=[
        pltpu.VMEM((2, C), jnp.int32),
        pltpu.VMEM((2, C), jnp.int32),
        pltpu.VMEM((C, HQ), jnp.float32),
        pltpu.VMEM((C, 2 * H), jnp.float32),
        pltpu.VMEM((C, H), jnp.float32),
        pltpu.VMEM((C,), jnp.float32),
        pltpu.VMEM_SHARED((NP, H), jnp.float32),
        pltpu.VMEM_SHARED((NP,), jnp.float32),
        pltpu.SemaphoreType.DMA,
        pltpu.SemaphoreType.DMA,
        pltpu.SemaphoreType.DMA,
        pltpu.SemaphoreType.DMA,
        pltpu.SemaphoreType.DMA,
        ],
    )


def _eu_body_with_dvec(src_h, dst_h, a_h, b_h, t_h, wv_h,
                       ea_o, rm_o, dv_o,
                       sidx, didx, abf, bbf, tbf, dvb, rmb, wvb, sem):
    cid = lax.axis_index("c")
    sid = lax.axis_index("s")
    wid = cid * NS + sid
    pltpu.sync_copy(wv_h, wvb)
    lanes = lax.iota(jnp.int32, 16)
    perms = [lanes ^ sh for sh in (8, 4, 2, 1)]

    def chunk(ci, rm):
        base = wid * EPW + ci * C
        pltpu.sync_copy(src_h.at[pl.ds(base, C)], sidx)
        pltpu.sync_copy(dst_h.at[pl.ds(base, C)], didx)
        cp_a = pltpu.async_copy(a_h.at[sidx], abf, sem)
        cp_b = pltpu.async_copy(b_h.at[didx], bbf, sem)
        cp_t = pltpu.async_copy(t_h.at[pl.ds(base, C)], tbf, sem)
        cp_a.wait()
        cp_b.wait()
        cp_t.wait()

        def group(g, rm2):
            dvv = jnp.zeros((16,), jnp.float32)
            for j in range(16):
                e = g * 16 + j
                sq = jnp.zeros((16,), jnp.float32)
                dv = jnp.zeros((16,), jnp.float32)
                for t in range(8):
                    sl = pl.ds(t * 16, 16)
                    z = abf[e, sl] + bbf[e, sl] + tbf[e, sl]
                    z = jnp.where(z > 0, z, z * jnp.float32(0.01))
                    tbf[e, sl] = z
                    sq = sq + z * z
                    dv = dv + z * wvb[sl]
                for perm in perms:
                    sq = sq + sq.at[perm].get(mode="promise_in_bounds")
                    dv = dv + dv.at[perm].get(mode="promise_in_bounds")
                rm2 = jnp.maximum(rm2, sq)
                dvv = jnp.where(lanes == j, dv, dvv)
            dvb[pl.ds(g * 16, 16)] = dvv
            return rm2

        rm = lax.fori_loop(0, C // 16, group, rm, unroll=False)
        pltpu.sync_copy(tbf, ea_o.at[pl.ds(base, C)])
        pltpu.sync_copy(dvb, dv_o.at[pl.ds(base, C)])
        return rm

    rm = lax.fori_loop(0, NCHUNK, chunk, jnp.zeros((16,), jnp.float32),
                       unroll=False)
    rmb[...] = rm
    pltpu.sync_copy(rmb, rm_o.at[cid, sid])


@functools.cache
def _eu_pass():
    return pl.kernel(
        _eu_body_with_dvec,
        out_type=[jax.ShapeDtypeStruct((E, H), jnp.float32),
                  jax.ShapeDtypeStruct((NC, NS, 16), jnp.float32),
                  jax.ShapeDtypeStruct((E,), jnp.float32)],
        mesh=_mesh(),
        compiler_params=pltpu.CompilerParams(use_tc_tiling_on_sc=False,
                                             needs_layout_passes=False),
        scratch_types=[
            pltpu.VMEM((C,), jnp.int32),
            pltpu.VMEM((C,), jnp.int32),
            pltpu.VMEM((C, H), jnp.float32),
            pltpu.VMEM((C, H), jnp.float32),
            pltpu.VMEM((C, H), jnp.float32),
            pltpu.VMEM((C,), jnp.float32),
            pltpu.VMEM((16,), jnp.float32),
            pltpu.VMEM((H,), jnp.float32),
            pltpu.SemaphoreType.DMA,
        ],
    )


def _final_body(src_h, dst_h, g1_h, g2_h, dv_h, out_o,
                sidx, didx, g1b, g2b, dvb, ob, sem):
    cid = lax.axis_index("c")
    sid = lax.axis_index("s")
    wid = cid * NS + sid

    def chunk(ci, carry):
        base = wid * EPW + ci * C
        pltpu.sync_copy(src_h.at[pl.ds(base, C)], sidx)
        pltpu.sync_copy(dst_h.at[pl.ds(base, C)], didx)
        cp_a = pltpu.async_copy(g1_h.at[sidx], g1b, sem)
        cp_b = pltpu.async_copy(g2_h.at[didx], g2b, sem)
        cp_d = pltpu.async_copy(dv_h.at[pl.ds(base, C)], dvb, sem)
        cp_a.wait()
        cp_b.wait()
        cp_d.wait()

        def group(g, c2):
            sl = pl.ds(g * 16, 16)
            z = g1b[sl] + g2b[sl] + dvb[sl]
            ob[sl] = 1.0 / (1.0 + jnp.exp(-z))
            return c2

        lax.fori_loop(0, C // 16, group, 0, unroll=False)
        pltpu.sync_copy(ob, out_o.at[pl.ds(base, C)])
        return carry

    lax.fori_loop(0, NCHUNK, chunk, 0, unroll=False)


@functools.cache
def _final_pass():
    return pl.kernel(
        _final_body,
        out_type=jax.ShapeDtypeStruct((E,), jnp.float32),
        mesh=_mesh(),
        compiler_params=pltpu.CompilerParams(use_tc_tiling_on_sc=False,
                                             needs_layout_passes=False),
        scratch_types=[
            pltpu.VMEM((C,), jnp.int32),
            pltpu.VMEM((C,), jnp.int32),
            pltpu.VMEM((C,), jnp.float32),
            pltpu.VMEM((C,), jnp.float32),
            pltpu.VMEM((C,), jnp.float32),
            pltpu.VMEM((C,), jnp.float32),
            pltpu.SemaphoreType.DMA,
        ],
    )


def _conv(h, src, dst, eaW, AW, p):
    q = h @ p["Wq"] + p["bq"]
    k = h @ p["Wk"] + p["bk"]
    v = h @ p["Wv"] + p["bv"]
    kv = jnp.concatenate([k, v], axis=1)
    K = jnp.max(jnp.linalg.norm(k, axis=1))
    s = jnp.linalg.norm(q, axis=1) * (K + AW) * RSQRT_H
    qpad = jnp.concatenate(
        [q * RSQRT_H, -s[:, None], jnp.zeros((N, HQ - H - 1), jnp.float32)], axis=1)
    z128 = jnp.zeros((NP, H), jnp.float32)
    z1 = jnp.zeros((NP,), jnp.float32)
    acc, den = _conv_edge_pass()(src, dst, qpad, kv, eaW, z128, z1)
    num = acc[0, :N] + acc[1, :N]
    denom = den[0, :N] + den[1, :N]
    out = jnp.where(denom[:, None] > 0, num / denom[:, None], 0.0)
    return out + h @ p["Ws"] + p["bs"]


def kernel(x, edge_attr, edge_index, params):
    p = params
    src, dst = edge_index[0], edge_index[1]
    h = x @ p["node_W"] + p["node_b"]

    c1 = p["conv1"]
    eaW1 = edge_attr @ (p["edge_W"] @ c1["We"]) + p["edge_b"] @ c1["We"]
    AW1 = jnp.max(jnp.linalg.norm(eaW1, axis=1))
    h = jax.nn.leaky_relu(_conv(h, src, dst, eaW1, AW1, c1))

    w = p["eu3_W"] @ p["fc_W"]
    c = p["eu3_b"] @ p["fc_W"] + p["fc_b"]

    A1 = h @ p["eu1_W"][:H]
    B1 = h @ p["eu1_W"][H:2 * H]
    T1 = edge_attr @ (p["edge_W"] @ p["eu1_W"][2 * H:]) + (
        p["edge_b"] @ p["eu1_W"][2 * H:] + p["eu1_b"])
    ea2, rm1, _ = _eu_pass()(src, dst, A1, B1, T1, jnp.zeros((H,), jnp.float32))
    Aea2 = jnp.sqrt(jnp.max(rm1))

    c2 = p["conv2"]
    eaW2 = ea2 @ c2["We"]
    AW2 = Aea2 * jnp.linalg.norm(c2["We"])
    h = jax.nn.leaky_relu(_conv(h, src, dst, eaW2, AW2, c2))

    A2 = h @ p["eu2_W"][:H]
    B2 = h @ p["eu2_W"][H:2 * H]
    T2 = ea2 @ p["eu2_W"][2 * H:] + p["eu2_b"]
    ea3, rm2, dvec = _eu_pass()(src, dst, A2, B2, T2, w[2 * H:, 0])
    Aea3 = jnp.sqrt(jnp.max(rm2))

    c3 = p["conv3"]
    eaW3 = ea3 @ c3["We"]
    AW3 = Aea3 * jnp.linalg.norm(c3["We"])
    h = _conv(h, src, dst, eaW3, AW3, c3)

    g1 = (h @ w[:H])[:, 0]
    g2 = (h @ w[H:2 * H])[:, 0] + c[0]
    out = _final_pass()(src, dst, g1, g2, dvec)
    return out[:, None]
